# Initial kernel scaffold; baseline (speedup 1.0000x reference)
import functools

import jax
import jax.numpy as jnp
from jax.experimental import pallas as pl
from jax.experimental.pallas import tpu as pltpu

_ALPHA = 0.25


def _focal_kernel(ann_ref, ax0_ref, ay0_ref, ax1_ref, ay1_ref,
                  cls_ref, r0_ref, r1_ref, r2_ref, r3_ref,
                  out_ref, acc_ref, *, blka, num_blocks, num_anchors,
                  num_classes, num_boxes):
    i = pl.program_id(0)

    @pl.when(i == 0)
    def _init():
        acc_ref[...] = jnp.zeros_like(acc_ref)

    ax0 = ax0_ref[...][None, :]
    ay0 = ay0_ref[...][None, :]
    ax1 = ax1_ref[...][None, :]
    ay1 = ay1_ref[...][None, :]
    aw = ax1 - ax0
    ah = ay1 - ay0
    acx = ax0 + 0.5 * aw
    acy = ay0 + 0.5 * ah
    area_a = aw * ah

    ann = ann_ref[...]
    nb = ann.shape[1]

    best = jnp.full((nb, blka), -1.0, dtype=jnp.float32)
    bcx = jnp.zeros_like(best)
    bcy = jnp.zeros_like(best)
    bwc = jnp.ones_like(best)
    bhc = jnp.ones_like(best)
    blab = jnp.zeros_like(best)

    for m in range(num_boxes):
        bx0 = ann[0, :, m][:, None]
        by0 = ann[1, :, m][:, None]
        bx1 = ann[2, :, m][:, None]
        by1 = ann[3, :, m][:, None]
        area_b = ann[4, :, m][:, None]
        iw = jnp.minimum(ax1, bx1) - jnp.maximum(ax0, bx0)
        ih = jnp.minimum(ay1, by1) - jnp.maximum(ay0, by0)
        iw = jnp.maximum(iw, 0.0)
        ih = jnp.maximum(ih, 0.0)
        inter = iw * ih
        ua = (area_a + area_b) - inter
        ua = jnp.maximum(ua, 1e-8)
        iou = inter / ua
        upd = iou > best
        best = jnp.where(upd, iou, best)
        bcx = jnp.where(upd, ann[5, :, m][:, None], bcx)
        bcy = jnp.where(upd, ann[6, :, m][:, None], bcy)
        bwc = jnp.where(upd, ann[7, :, m][:, None], bwc)
        bhc = jnp.where(upd, ann[8, :, m][:, None], bhc)
        blab = jnp.where(upd, ann[9, :, m][:, None], blab)

    aid = jax.lax.broadcasted_iota(jnp.int32, (1, blka), 1) + i * blka
    valid = aid < num_anchors
    pos = jnp.logical_and(best >= 0.5, valid)
    neg = jnp.logical_and(best < 0.4, valid)

    c = cls_ref[...]
    l = jnp.log(1.0 - c)
    s = (c * c) * l
    k3 = blab[:, :, None]
    cid = jax.lax.broadcasted_iota(jnp.float32, (1, 1, num_classes), 2)
    msel = cid == k3
    srow = jnp.sum(s, axis=2)
    ck = jnp.sum(jnp.where(msel, c, 0.0), axis=2)
    sk = jnp.sum(jnp.where(msel, s, 0.0), axis=2)
    ck = jnp.clip(ck, 1e-6, 1.0 - 1e-6)
    pos_term = _ALPHA * (1.0 - ck) * (1.0 - ck) * (-jnp.log(ck))
    loss_pos = -(1.0 - _ALPHA) * (srow - sk) + pos_term
    loss_neg = -(1.0 - _ALPHA) * srow
    cls_contrib = jnp.where(pos, loss_pos, jnp.where(neg, loss_neg, 0.0))

    safe_aw = jnp.maximum(aw, 1e-6)
    safe_ah = jnp.maximum(ah, 1e-6)
    t0 = ((bcx - acx) / safe_aw) / 0.1
    t1 = ((bcy - acy) / safe_ah) / 0.1
    t2 = jnp.log(bwc / safe_aw) / 0.2
    t3 = jnp.log(bhc / safe_ah) / 0.2
    rsum = jnp.zeros_like(best)
    for t, rref in ((t0, r0_ref), (t1, r1_ref), (t2, r2_ref), (t3, r3_ref)):
        diff = jnp.abs(t - rref[...])
        rsum += jnp.where(diff <= 1.0 / 9.0, 0.5 * 9.0 * diff * diff,
                          diff - 0.5 / 9.0)
    rgs_contrib = jnp.where(pos, rsum, 0.0)

    acc_ref[0, :] += jnp.sum(cls_contrib, axis=1)
    acc_ref[1, :] += jnp.sum(rgs_contrib, axis=1)
    acc_ref[2, :] += jnp.sum(jnp.where(pos, 1.0, 0.0), axis=1)

    @pl.when(i == num_blocks - 1)
    def _fin():
        npos = acc_ref[2, :]
        cls_out = acc_ref[0, :] / jnp.maximum(npos, 1.0)
        rgs_out = jnp.where(npos > 0.0,
                            acc_ref[1, :] / jnp.maximum(npos * 4.0, 1.0), 0.0)
        out_ref[0, :] = cls_out
        out_ref[1, :] = rgs_out


def kernel(classifications, regressions, anchors, annotations):
    B, A, C = classifications.shape
    M = annotations.shape[1]
    blka = 2048 if A >= 2048 else ((A + 127) // 128) * 128
    num_blocks = pl.cdiv(A, blka)

    a = anchors[0]
    ax0, ay0, ax1, ay1 = a[:, 0], a[:, 1], a[:, 2], a[:, 3]

    bx0 = annotations[:, :, 0]
    by0 = annotations[:, :, 1]
    bx1 = annotations[:, :, 2]
    by1 = annotations[:, :, 3]
    bw = bx1 - bx0
    bh = by1 - by0
    ann = jnp.stack([
        bx0, by0, bx1, by1,
        bw * bh,
        bx0 + 0.5 * bw,
        by0 + 0.5 * bh,
        jnp.clip(bw, 1.0, None),
        jnp.clip(bh, 1.0, None),
        annotations[:, :, 4],
    ])

    r0 = regressions[:, :, 0]
    r1 = regressions[:, :, 1]
    r2 = regressions[:, :, 2]
    r3 = regressions[:, :, 3]

    body = functools.partial(
        _focal_kernel, blka=blka, num_blocks=num_blocks, num_anchors=A,
        num_classes=C, num_boxes=M)
    out = pl.pallas_call(
        body,
        grid=(num_blocks,),
        in_specs=[
            pl.BlockSpec((10, B, M), lambda i: (0, 0, 0)),
            pl.BlockSpec((blka,), lambda i: (i,)),
            pl.BlockSpec((blka,), lambda i: (i,)),
            pl.BlockSpec((blka,), lambda i: (i,)),
            pl.BlockSpec((blka,), lambda i: (i,)),
            pl.BlockSpec((B, blka, C), lambda i: (0, i, 0)),
            pl.BlockSpec((B, blka), lambda i: (0, i)),
            pl.BlockSpec((B, blka), lambda i: (0, i)),
            pl.BlockSpec((B, blka), lambda i: (0, i)),
            pl.BlockSpec((B, blka), lambda i: (0, i)),
        ],
        out_specs=pl.BlockSpec((2, B), lambda i: (0, 0)),
        out_shape=jax.ShapeDtypeStruct((2, B), jnp.float32),
        scratch_shapes=[pltpu.VMEM((3, B), jnp.float32)],
    )(ann, ax0, ay0, ax1, ay1, classifications, r0, r1, r2, r3)
    return out


# speedup vs baseline: 4.2384x; 4.2384x over previous
import functools

import jax
import jax.numpy as jnp
from jax.experimental import pallas as pl
from jax.experimental.pallas import tpu as pltpu

_ALPHA = 0.25


def _focal_kernel(ann_ref, ax0_ref, ay0_ref, ax1_ref, ay1_ref,
                  cls_ref, r0_ref, r1_ref, r2_ref, r3_ref,
                  out_ref, acc_ref, *, blka, num_blocks, num_anchors,
                  num_classes, num_boxes):
    i = pl.program_id(0)

    @pl.when(i == 0)
    def _init():
        acc_ref[...] = jnp.zeros_like(acc_ref)

    ax0 = ax0_ref[...][None, :]
    ay0 = ay0_ref[...][None, :]
    ax1 = ax1_ref[...][None, :]
    ay1 = ay1_ref[...][None, :]
    aw = ax1 - ax0
    ah = ay1 - ay0
    acx = ax0 + 0.5 * aw
    acy = ay0 + 0.5 * ah
    area_a = aw * ah

    ann = ann_ref[...]
    nb = ann.shape[1]

    best = jnp.full((nb, blka), -1.0, dtype=jnp.float32)
    bcx = jnp.zeros_like(best)
    bcy = jnp.zeros_like(best)
    bwc = jnp.ones_like(best)
    bhc = jnp.ones_like(best)
    blab = jnp.zeros_like(best)

    for m in range(num_boxes):
        bx0 = ann[0, :, m][:, None]
        by0 = ann[1, :, m][:, None]
        bx1 = ann[2, :, m][:, None]
        by1 = ann[3, :, m][:, None]
        area_b = ann[4, :, m][:, None]
        iw = jnp.minimum(ax1, bx1) - jnp.maximum(ax0, bx0)
        ih = jnp.minimum(ay1, by1) - jnp.maximum(ay0, by0)
        iw = jnp.maximum(iw, 0.0)
        ih = jnp.maximum(ih, 0.0)
        inter = iw * ih
        ua = (area_a + area_b) - inter
        ua = jnp.maximum(ua, 1e-8)
        iou = inter / ua
        upd = iou > best
        best = jnp.where(upd, iou, best)
        bcx = jnp.where(upd, ann[5, :, m][:, None], bcx)
        bcy = jnp.where(upd, ann[6, :, m][:, None], bcy)
        bwc = jnp.where(upd, ann[7, :, m][:, None], bwc)
        bhc = jnp.where(upd, ann[8, :, m][:, None], bhc)
        blab = jnp.where(upd, ann[9, :, m][:, None], blab)

    aid = jax.lax.broadcasted_iota(jnp.int32, (1, blka), 1) + i * blka
    valid = aid < num_anchors
    pos = jnp.logical_and(best >= 0.5, valid)
    neg = jnp.logical_and(best < 0.4, valid)

    c = cls_ref[...]
    l = jnp.log(1.0 - c)
    s = (c * c) * l
    k3 = blab.astype(jnp.int32)[:, :, None]
    cid = jax.lax.broadcasted_iota(jnp.int32, (1, 1, num_classes), 2)
    msel = cid == k3
    srow = jnp.sum(s, axis=2)
    ck = jnp.sum(jnp.where(msel, c, 0.0), axis=2)
    sk = jnp.sum(jnp.where(msel, s, 0.0), axis=2)
    ck = jnp.clip(ck, 1e-6, 1.0 - 1e-6)
    pos_term = _ALPHA * (1.0 - ck) * (1.0 - ck) * (-jnp.log(ck))
    loss_pos = -(1.0 - _ALPHA) * (srow - sk) + pos_term
    loss_neg = -(1.0 - _ALPHA) * srow
    cls_contrib = jnp.where(pos, loss_pos, jnp.where(neg, loss_neg, 0.0))

    safe_aw = jnp.maximum(aw, 1e-6)
    safe_ah = jnp.maximum(ah, 1e-6)
    t0 = ((bcx - acx) / safe_aw) / 0.1
    t1 = ((bcy - acy) / safe_ah) / 0.1
    t2 = jnp.log(bwc / safe_aw) / 0.2
    t3 = jnp.log(bhc / safe_ah) / 0.2
    rsum = jnp.zeros_like(best)
    for t, rref in ((t0, r0_ref), (t1, r1_ref), (t2, r2_ref), (t3, r3_ref)):
        diff = jnp.abs(t - rref[...])
        rsum += jnp.where(diff <= 1.0 / 9.0, 0.5 * 9.0 * diff * diff,
                          diff - 0.5 / 9.0)
    rgs_contrib = jnp.where(pos, rsum, 0.0)

    acc_ref[0, :] += jnp.sum(cls_contrib, axis=1)
    acc_ref[1, :] += jnp.sum(rgs_contrib, axis=1)
    acc_ref[2, :] += jnp.sum(jnp.where(pos, 1.0, 0.0), axis=1)

    @pl.when(i == num_blocks - 1)
    def _fin():
        npos = acc_ref[2, :]
        cls_out = acc_ref[0, :] / jnp.maximum(npos, 1.0)
        rgs_out = jnp.where(npos > 0.0,
                            acc_ref[1, :] / jnp.maximum(npos * 4.0, 1.0), 0.0)
        out_ref[0, :] = cls_out
        out_ref[1, :] = rgs_out


def kernel(classifications, regressions, anchors, annotations):
    B, A, C = classifications.shape
    M = annotations.shape[1]
    blka = 2048 if A >= 2048 else ((A + 127) // 128) * 128
    num_blocks = pl.cdiv(A, blka)

    a = anchors[0]
    ax0, ay0, ax1, ay1 = a[:, 0], a[:, 1], a[:, 2], a[:, 3]

    bx0 = annotations[:, :, 0]
    by0 = annotations[:, :, 1]
    bx1 = annotations[:, :, 2]
    by1 = annotations[:, :, 3]
    bw = bx1 - bx0
    bh = by1 - by0
    ann = jnp.stack([
        bx0, by0, bx1, by1,
        bw * bh,
        bx0 + 0.5 * bw,
        by0 + 0.5 * bh,
        jnp.clip(bw, 1.0, None),
        jnp.clip(bh, 1.0, None),
        annotations[:, :, 4],
    ])

    r0 = regressions[:, :, 0]
    r1 = regressions[:, :, 1]
    r2 = regressions[:, :, 2]
    r3 = regressions[:, :, 3]

    body = functools.partial(
        _focal_kernel, blka=blka, num_blocks=num_blocks, num_anchors=A,
        num_classes=C, num_boxes=M)
    out = pl.pallas_call(
        body,
        grid=(num_blocks,),
        in_specs=[
            pl.BlockSpec((10, B, M), lambda i: (0, 0, 0)),
            pl.BlockSpec((blka,), lambda i: (i,)),
            pl.BlockSpec((blka,), lambda i: (i,)),
            pl.BlockSpec((blka,), lambda i: (i,)),
            pl.BlockSpec((blka,), lambda i: (i,)),
            pl.BlockSpec((B, blka, C), lambda i: (0, i, 0)),
            pl.BlockSpec((B, blka), lambda i: (0, i)),
            pl.BlockSpec((B, blka), lambda i: (0, i)),
            pl.BlockSpec((B, blka), lambda i: (0, i)),
            pl.BlockSpec((B, blka), lambda i: (0, i)),
        ],
        out_specs=pl.BlockSpec((2, B), lambda i: (0, 0)),
        out_shape=jax.ShapeDtypeStruct((2, B), jnp.float32),
        scratch_shapes=[pltpu.VMEM((3, B), jnp.float32)],
    )(ann, ax0, ay0, ax1, ay1, classifications, r0, r1, r2, r3)
    return out
